# baseline (speedup 1.0000x reference)
import jax
import jax.numpy as jnp
from jax import lax
from jax.experimental import pallas as pl
from jax.experimental.pallas import tpu as pltpu
from jax.experimental.pallas import tpu_sc as plsc

N = 10000
E = 320000
D = 128
NCLS = 40

NC = 2
NS = 16
NW = NC * NS
WIN = 128
EPW = E // NW

NB = 4
RANGE = 2528
N_PAD = NB * RANGE
CAPW = 22
CAP = CAPW * WIN
ACC_ROWS = RANGE + 160
DUMMY = 160
NCH = 2 * CAPW

_mesh = plsc.VectorSubcoreMesh(core_axis_name="c", subcore_axis_name="s")
_params = pltpu.CompilerParams(needs_layout_passes=False)


def _part_body(ei_hbm, srcp_hbm, dstp_hbm, cnt_hbm, deg_hbm,
               sin, din, sbuf, dbuf, cnts, hist, sem):
    c = lax.axis_index("c")
    s = lax.axis_index("s")
    w = s * NC + c

    iota = lax.iota(jnp.int32, 16)

    def pre(r, _):
        for k in range(8):
            v = r * 128 + k * 16 + iota
            for b in range(NB):
                sbuf[b][r, pl.ds(k * 16, 16)] = v
                dbuf[b][r, pl.ds(k * 16, 16)] = RANGE + v % DUMMY
        return 0

    lax.fori_loop(0, CAPW, pre, 0)

    def zhist(r, _):
        hist[pl.ds(r * 16, 16)] = jnp.zeros((16,), jnp.float32)
        return 0

    lax.fori_loop(0, N_PAD // 16, zhist, 0)

    base = w * (EPW - 16)
    pltpu.sync_copy(ei_hbm.at[0].at[pl.ds(base, EPW - 16)],
                    sin.at[pl.ds(0, EPW - 16)])
    pltpu.sync_copy(ei_hbm.at[1].at[pl.ds(base, EPW - 16)],
                    din.at[pl.ds(0, EPW - 16)])

    @pl.when(w < 4)
    def _():
        xb = 32 * (EPW - 16) + w * 128
        pltpu.sync_copy(ei_hbm.at[0].at[pl.ds(xb, 128)],
                        sin.at[pl.ds(EPW - 16, 128)])
        pltpu.sync_copy(ei_hbm.at[1].at[pl.ds(xb, 128)],
                        din.at[pl.ds(EPW - 16, 128)])

    nv = jnp.where(w < 4, (EPW - 16 + 128) // 16, (EPW - 16) // 16)

    ones16 = jnp.ones((16,), jnp.float32)

    def step(k, offs):
        sv = sin[pl.ds(k * 16, 16)]
        dv = din[pl.ds(k * 16, 16)]
        plsc.addupdate_scatter(hist, [dv], ones16)
        new_offs = []
        for b in range(NB):
            lo = b * RANGE
            m = (dv >= lo) & (dv < lo + RANGE) if b else (dv < RANGE)
            mi = m.astype(jnp.int32)
            pos = offs[b] + plsc.cumsum(mi) - 1
            pr = lax.shift_right_logical(pos, 7)
            pc = lax.bitwise_and(pos, 127)
            plsc.store_scatter(sbuf[b], [pr, pc], sv, mask=m)
            plsc.store_scatter(dbuf[b], [pr, pc], dv - lo, mask=m)
            cnt = jnp.sum(mi, axis=0)
            new_offs.append(jnp.minimum(offs[b] + cnt, CAP - 16))
        return tuple(new_offs)

    offs = lax.fori_loop(0, nv, step, (0, 0, 0, 0))

    for b in range(NB):
        pltpu.sync_copy(sbuf[b], srcp_hbm.at[w, b])
        pltpu.sync_copy(dbuf[b], dstp_hbm.at[w, b])
        cv = jnp.full((16,), offs[b], jnp.int32)
        cnts[pl.ds(b * 16, 16)] = cv
    pltpu.sync_copy(cnts, cnt_hbm.at[w])
    pltpu.sync_copy(hist, deg_hbm.at[w])


_sc_partition = pl.kernel(
    _part_body,
    out_type=[
        jax.ShapeDtypeStruct((NW, NB, CAPW, WIN), jnp.int32),
        jax.ShapeDtypeStruct((NW, NB, CAPW, WIN), jnp.int32),
        jax.ShapeDtypeStruct((NW, NB * 16), jnp.int32),
        jax.ShapeDtypeStruct((NW, N_PAD), jnp.float32),
    ],
    mesh=_mesh,
    scratch_types=[
        pltpu.VMEM((10112,), jnp.int32),
        pltpu.VMEM((10112,), jnp.int32),
        [pltpu.VMEM((CAPW, WIN), jnp.int32) for _ in range(NB)],
        [pltpu.VMEM((CAPW, WIN), jnp.int32) for _ in range(NB)],
        pltpu.VMEM((NB * 16,), jnp.int32),
        pltpu.VMEM((N_PAD,), jnp.float32),
        pltpu.SemaphoreType.DMA,
    ],
    compiler_params=_params,
)


def _agg_body(t_hbm, srcp_hbm, dstp_hbm, cnt_hbm, out_hbm,
              srcv, dstv, rows, cbuf, zbig, acc, gsem):
    c = lax.axis_index("c")
    s = lax.axis_index("s")

    def zrow(r, _):
        for k in range(D // 16):
            zbig[r, pl.ds(k * 16, 16)] = jnp.zeros((16,), jnp.float32)
        return 0

    lax.fori_loop(0, 128, zrow, 0)

    pltpu.sync_copy(cnt_hbm.at[2 * s], cbuf.at[0])
    pltpu.sync_copy(cnt_hbm.at[2 * s + 1], cbuf.at[1])

    for p in range(2):
        b = 2 * p + c

        zb = s * (ACC_ROWS // NS)
        pltpu.sync_copy(zbig, acc.at[pl.ds(zb, 128)])
        pltpu.sync_copy(zbig.at[pl.ds(0, 40)], acc.at[pl.ds(zb + 128, 40)])
        plsc.subcore_barrier()

        pltpu.sync_copy(srcp_hbm.at[2 * s, b], srcv.at[pl.ds(0, CAPW)])
        pltpu.sync_copy(srcp_hbm.at[2 * s + 1, b], srcv.at[pl.ds(CAPW, CAPW)])
        pltpu.sync_copy(dstp_hbm.at[2 * s, b], dstv.at[pl.ds(0, CAPW)])
        pltpu.sync_copy(dstp_hbm.at[2 * s + 1, b], dstv.at[pl.ds(CAPW, CAPW)])

        nw0 = lax.shift_right_logical(
            jnp.max(cbuf[0, pl.ds(b * 16, 16)], axis=0) + WIN - 1, 7)
        nw1 = lax.shift_right_logical(
            jnp.max(cbuf[1, pl.ds(b * 16, 16)], axis=0) + WIN - 1, 7)
        nt = nw0 + nw1

        def jj(j):
            return jnp.where(j < nw0, j, j - nw0 + CAPW)

        @pl.when(nt > 0)
        def _():
            pltpu.async_copy(t_hbm.at[srcv.at[jj(0)]], rows[0], gsem[0])

        def step(i, _):
            j0 = 2 * i
            j1 = j0 + 1

            @pl.when(j1 < nt)
            def _():
                pltpu.async_copy(t_hbm.at[srcv.at[jj(j1)]], rows[1], gsem[1])

            pltpu.make_async_copy(
                t_hbm.at[srcv.at[jj(j0)]], rows[0], gsem[0]).wait()
            pltpu.sync_copy(rows[0], acc.at[dstv.at[jj(j0)]], add=True)

            @pl.when(j0 + 2 < nt)
            def _():
                pltpu.async_copy(t_hbm.at[srcv.at[jj(j0 + 2)]], rows[0], gsem[0])

            @pl.when(j1 < nt)
            def _():
                pltpu.make_async_copy(
                    t_hbm.at[srcv.at[jj(j1)]], rows[1], gsem[1]).wait()
                pltpu.sync_copy(rows[1], acc.at[dstv.at[jj(j1)]], add=True)

            return 0

        lax.fori_loop(0, (nt + 1) // 2, step, 0)
        plsc.subcore_barrier()

        gb = b * RANGE + 160 * s

        @pl.when(s < NS - 1)
        def _():
            pltpu.sync_copy(acc.at[pl.ds(160 * s, 160)],
                            out_hbm.at[pl.ds(gb, 160)])

        @pl.when(s == NS - 1)
        def _():
            pltpu.sync_copy(acc.at[pl.ds(160 * s, 128)],
                            out_hbm.at[pl.ds(gb, 128)])

        plsc.subcore_barrier()


_sc_agg = pl.kernel(
    _agg_body,
    out_type=jax.ShapeDtypeStruct((N_PAD, D), jnp.float32),
    mesh=_mesh,
    scratch_types=[
        pltpu.VMEM((NCH, WIN), jnp.int32),
        pltpu.VMEM((NCH, WIN), jnp.int32),
        [pltpu.VMEM((WIN, D), jnp.float32) for _ in range(2)],
        pltpu.VMEM((2, NB * 16), jnp.int32),
        pltpu.VMEM((128, D), jnp.float32),
        pltpu.VMEM_SHARED((ACC_ROWS, D), jnp.float32),
        [pltpu.SemaphoreType.DMA for _ in range(2)],
    ],
    compiler_params=_params,
)


def _k1_body(x_ref, w_ref, t0_ref):
    t0_ref[...] = jnp.dot(x_ref[...], w_ref[...],
                          preferred_element_type=jnp.float32)


def _kdeg_body(degp_ref, o_ref):
    d = jnp.sum(degp_ref[...], axis=0)
    o_ref[...] = (1.0 / jnp.maximum(d, 1.0)).reshape(1, N_PAD)


def _kself_body(h_ref, w_ref, b_ref, o_ref):
    o_ref[...] = jnp.dot(h_ref[...], w_ref[...],
                         preferred_element_type=jnp.float32) + b_ref[...]


def _k3_body(xs_ref, agg_ref, dinv_ref, wn1_ref, h1_ref, t1_ref):
    hn = agg_ref[:N, :] * dinv_ref[...]
    h1 = jnp.maximum(xs_ref[...] + hn, 0.0)
    h1_ref[...] = h1
    t1_ref[...] = jnp.dot(h1, wn1_ref[...], preferred_element_type=jnp.float32)


def _k5_body(hs_ref, agg_ref, dinv_ref, fcw_ref, fcb_ref, o_ref):
    hn = agg_ref[:N, :] * dinv_ref[...]
    h2 = hs_ref[...] + hn
    o_ref[...] = jnp.dot(h2, fcw_ref[...],
                         preferred_element_type=jnp.float32) + fcb_ref[...]


def kernel(x, edge_index, W_self0, W_neigh0, b0, W_self1, W_neigh1, b1,
           fc_W, fc_b):
    b0r = b0.reshape(1, D)
    b1r = b1.reshape(1, D)
    fcbr = fc_b.reshape(1, NCLS)

    t0 = pl.pallas_call(
        _k1_body,
        out_shape=jax.ShapeDtypeStruct((N, D), jnp.float32),
    )(x, W_neigh0)

    srcp, dstp, cntp, degp = _sc_partition(edge_index.astype(jnp.int32))

    dinv2d = pl.pallas_call(
        _kdeg_body,
        out_shape=jax.ShapeDtypeStruct((1, N_PAD), jnp.float32),
    )(degp)
    dinv_col = dinv2d.reshape(N_PAD, 1)[:N]

    agg0 = _sc_agg(t0, srcp, dstp, cntp)

    xs = pl.pallas_call(
        _kself_body,
        out_shape=jax.ShapeDtypeStruct((N, D), jnp.float32),
    )(x, W_self0, b0r)

    h1, t1 = pl.pallas_call(
        _k3_body,
        out_shape=[
            jax.ShapeDtypeStruct((N, D), jnp.float32),
            jax.ShapeDtypeStruct((N, D), jnp.float32),
        ],
    )(xs, agg0, dinv_col, W_neigh1)

    agg1 = _sc_agg(t1, srcp, dstp, cntp)

    hs = pl.pallas_call(
        _kself_body,
        out_shape=jax.ShapeDtypeStruct((N, D), jnp.float32),
    )(h1, W_self1, b1r)

    out = pl.pallas_call(
        _k5_body,
        out_shape=jax.ShapeDtypeStruct((N, NCLS), jnp.float32),
    )(hs, agg1, dinv_col, fc_W, fcbr)

    return out


# speedup vs baseline: 1.0114x; 1.0114x over previous
import jax
import jax.numpy as jnp
from jax import lax
from jax.experimental import pallas as pl
from jax.experimental.pallas import tpu as pltpu
from jax.experimental.pallas import tpu_sc as plsc

N = 10000
E = 320000
D = 128
NCLS = 40

NC = 2
NS = 16
NW = NC * NS
WIN = 128
EPW = E // NW

NB = 4
RANGE = 2528
N_PAD = NB * RANGE
CAPW = 22
CAP = CAPW * WIN
ACC_ROWS = RANGE + 160
DUMMY = 160
NCH = 2 * CAPW

_mesh = plsc.VectorSubcoreMesh(core_axis_name="c", subcore_axis_name="s")
_params = pltpu.CompilerParams(needs_layout_passes=False)


def _part_body(ei_hbm, srcp_hbm, dstp_hbm, cnt_hbm, deg_hbm,
               sin, din, sbuf, dbuf, cnts, hist, sem):
    c = lax.axis_index("c")
    s = lax.axis_index("s")
    w = s * NC + c

    iota = lax.iota(jnp.int32, 16)

    def pre(r, _):
        for k in range(8):
            v = r * 128 + k * 16 + iota
            for b in range(NB):
                sbuf[b][r, pl.ds(k * 16, 16)] = v
                dbuf[b][r, pl.ds(k * 16, 16)] = RANGE + v % DUMMY
        return 0

    lax.fori_loop(0, CAPW, pre, 0)

    def zhist(r, _):
        hist[pl.ds(r * 16, 16)] = jnp.zeros((16,), jnp.float32)
        return 0

    lax.fori_loop(0, N_PAD // 16, zhist, 0)

    base = w * (EPW - 16)
    pltpu.sync_copy(ei_hbm.at[0].at[pl.ds(base, EPW - 16)],
                    sin.at[pl.ds(0, EPW - 16)])
    pltpu.sync_copy(ei_hbm.at[1].at[pl.ds(base, EPW - 16)],
                    din.at[pl.ds(0, EPW - 16)])

    @pl.when(w < 4)
    def _():
        xb = 32 * (EPW - 16) + w * 128
        pltpu.sync_copy(ei_hbm.at[0].at[pl.ds(xb, 128)],
                        sin.at[pl.ds(EPW - 16, 128)])
        pltpu.sync_copy(ei_hbm.at[1].at[pl.ds(xb, 128)],
                        din.at[pl.ds(EPW - 16, 128)])

    nv = jnp.where(w < 4, (EPW - 16 + 128) // 16, (EPW - 16) // 16)

    ones16 = jnp.ones((16,), jnp.float32)

    def step(k, offs):
        sv = sin[pl.ds(k * 16, 16)]
        dv = din[pl.ds(k * 16, 16)]
        plsc.addupdate_scatter(hist, [dv], ones16)
        new_offs = []
        for b in range(NB):
            lo = b * RANGE
            m = (dv >= lo) & (dv < lo + RANGE) if b else (dv < RANGE)
            mi = m.astype(jnp.int32)
            pos = offs[b] + plsc.cumsum(mi) - 1
            pr = lax.shift_right_logical(pos, 7)
            pc = lax.bitwise_and(pos, 127)
            plsc.store_scatter(sbuf[b], [pr, pc], sv, mask=m)
            plsc.store_scatter(dbuf[b], [pr, pc], dv - lo, mask=m)
            cnt = jnp.sum(mi, axis=0)
            new_offs.append(jnp.minimum(offs[b] + cnt, CAP - 16))
        return tuple(new_offs)

    offs = lax.fori_loop(0, nv, step, (0, 0, 0, 0))

    for b in range(NB):
        pltpu.sync_copy(sbuf[b], srcp_hbm.at[w, b])
        pltpu.sync_copy(dbuf[b], dstp_hbm.at[w, b])
        cv = jnp.full((16,), offs[b], jnp.int32)
        cnts[pl.ds(b * 16, 16)] = cv
    pltpu.sync_copy(cnts, cnt_hbm.at[w])
    pltpu.sync_copy(hist, deg_hbm.at[w])


_sc_partition = pl.kernel(
    _part_body,
    out_type=[
        jax.ShapeDtypeStruct((NW, NB, CAPW, WIN), jnp.int32),
        jax.ShapeDtypeStruct((NW, NB, CAPW, WIN), jnp.int32),
        jax.ShapeDtypeStruct((NW, NB * 16), jnp.int32),
        jax.ShapeDtypeStruct((NW, N_PAD), jnp.float32),
    ],
    mesh=_mesh,
    scratch_types=[
        pltpu.VMEM((10112,), jnp.int32),
        pltpu.VMEM((10112,), jnp.int32),
        [pltpu.VMEM((CAPW, WIN), jnp.int32) for _ in range(NB)],
        [pltpu.VMEM((CAPW, WIN), jnp.int32) for _ in range(NB)],
        pltpu.VMEM((NB * 16,), jnp.int32),
        pltpu.VMEM((N_PAD,), jnp.float32),
        pltpu.SemaphoreType.DMA,
    ],
    compiler_params=_params,
)


def _agg_body(t_hbm, srcp_hbm, dstp_hbm, cnt_hbm, out_hbm,
              srcv, dstv, rows, cbuf, zbig, acc, gsem):
    c = lax.axis_index("c")
    s = lax.axis_index("s")

    def zrow(r, _):
        for k in range(D // 16):
            zbig[r, pl.ds(k * 16, 16)] = jnp.zeros((16,), jnp.float32)
        return 0

    lax.fori_loop(0, 128, zrow, 0)

    pltpu.sync_copy(cnt_hbm.at[2 * s], cbuf.at[0])
    pltpu.sync_copy(cnt_hbm.at[2 * s + 1], cbuf.at[1])

    for p in range(2):
        b = 2 * p + c

        zb = s * (ACC_ROWS // NS)
        pltpu.sync_copy(zbig, acc.at[pl.ds(zb, 128)])
        pltpu.sync_copy(zbig.at[pl.ds(0, 40)], acc.at[pl.ds(zb + 128, 40)])
        plsc.subcore_barrier()

        pltpu.sync_copy(srcp_hbm.at[2 * s, b], srcv.at[pl.ds(0, CAPW)])
        pltpu.sync_copy(srcp_hbm.at[2 * s + 1, b], srcv.at[pl.ds(CAPW, CAPW)])
        pltpu.sync_copy(dstp_hbm.at[2 * s, b], dstv.at[pl.ds(0, CAPW)])
        pltpu.sync_copy(dstp_hbm.at[2 * s + 1, b], dstv.at[pl.ds(CAPW, CAPW)])

        nw0 = lax.shift_right_logical(
            jnp.max(cbuf[0, pl.ds(b * 16, 16)], axis=0) + WIN - 1, 7)
        nw1 = lax.shift_right_logical(
            jnp.max(cbuf[1, pl.ds(b * 16, 16)], axis=0) + WIN - 1, 7)
        nt = nw0 + nw1

        def jj(j):
            return jnp.where(j < nw0, j, j - nw0 + CAPW)

        @pl.when(nt > 0)
        def _():
            pltpu.async_copy(t_hbm.at[srcv.at[jj(0)]], rows[0], gsem[0])

        def step(i, _):
            j0 = 2 * i
            j1 = j0 + 1

            @pl.when(j1 < nt)
            def _():
                pltpu.async_copy(t_hbm.at[srcv.at[jj(j1)]], rows[1], gsem[1])

            pltpu.make_async_copy(
                t_hbm.at[srcv.at[jj(j0)]], rows[0], gsem[0]).wait()
            pltpu.sync_copy(rows[0], acc.at[dstv.at[jj(j0)]], add=True)

            @pl.when(j0 + 2 < nt)
            def _():
                pltpu.async_copy(t_hbm.at[srcv.at[jj(j0 + 2)]], rows[0], gsem[0])

            @pl.when(j1 < nt)
            def _():
                pltpu.make_async_copy(
                    t_hbm.at[srcv.at[jj(j1)]], rows[1], gsem[1]).wait()
                pltpu.sync_copy(rows[1], acc.at[dstv.at[jj(j1)]], add=True)

            return 0

        lax.fori_loop(0, (nt + 1) // 2, step, 0)
        plsc.subcore_barrier()

        gb = b * RANGE + 160 * s

        @pl.when(s < NS - 1)
        def _():
            pltpu.sync_copy(acc.at[pl.ds(160 * s, 160)],
                            out_hbm.at[pl.ds(gb, 160)])

        @pl.when(s == NS - 1)
        def _():
            pltpu.sync_copy(acc.at[pl.ds(160 * s, 128)],
                            out_hbm.at[pl.ds(gb, 128)])

        plsc.subcore_barrier()


_sc_agg = pl.kernel(
    _agg_body,
    out_type=jax.ShapeDtypeStruct((N_PAD, D), jnp.float32),
    mesh=_mesh,
    scratch_types=[
        pltpu.VMEM((NCH, WIN), jnp.int32),
        pltpu.VMEM((NCH, WIN), jnp.int32),
        [pltpu.VMEM((WIN, D), jnp.float32) for _ in range(2)],
        pltpu.VMEM((2, NB * 16), jnp.int32),
        pltpu.VMEM((128, D), jnp.float32),
        pltpu.VMEM_SHARED((ACC_ROWS, D), jnp.float32),
        [pltpu.SemaphoreType.DMA for _ in range(2)],
    ],
    compiler_params=_params,
)


def _k1_body(x_ref, w_ref, t0_ref):
    t0_ref[...] = jnp.dot(x_ref[...], w_ref[...],
                          preferred_element_type=jnp.float32)


def _kdeg_body(degp_ref, o_ref):
    d = jnp.sum(degp_ref[...], axis=0)
    o_ref[...] = (1.0 / jnp.maximum(d, 1.0)).reshape(1, N_PAD)


def _k3_body(x_ref, agg_ref, dinv_ref, ws_ref, b_ref, wn1_ref,
             h1_ref, t1_ref):
    hn = agg_ref[:N, :] * dinv_ref[...]
    h1 = jnp.maximum(
        jnp.dot(x_ref[...], ws_ref[...], preferred_element_type=jnp.float32)
        + hn + b_ref[...], 0.0)
    h1_ref[...] = h1
    t1_ref[...] = jnp.dot(h1, wn1_ref[...], preferred_element_type=jnp.float32)


def _k5_body(h1_ref, agg_ref, dinv_ref, ws1_ref, b1_ref, fcw_ref, fcb_ref,
             o_ref):
    hn = agg_ref[:N, :] * dinv_ref[...]
    h2 = (jnp.dot(h1_ref[...], ws1_ref[...], preferred_element_type=jnp.float32)
          + hn + b1_ref[...])
    o_ref[...] = jnp.dot(h2, fcw_ref[...],
                         preferred_element_type=jnp.float32) + fcb_ref[...]


def kernel(x, edge_index, W_self0, W_neigh0, b0, W_self1, W_neigh1, b1,
           fc_W, fc_b):
    b0r = b0.reshape(1, D)
    b1r = b1.reshape(1, D)
    fcbr = fc_b.reshape(1, NCLS)

    t0 = pl.pallas_call(
        _k1_body,
        out_shape=jax.ShapeDtypeStruct((N, D), jnp.float32),
    )(x, W_neigh0)

    srcp, dstp, cntp, degp = _sc_partition(edge_index.astype(jnp.int32))

    dinv2d = pl.pallas_call(
        _kdeg_body,
        out_shape=jax.ShapeDtypeStruct((1, N_PAD), jnp.float32),
    )(degp)
    dinv_col = dinv2d.reshape(N_PAD, 1)[:N]

    agg0 = _sc_agg(t0, srcp, dstp, cntp)

    h1, t1 = pl.pallas_call(
        _k3_body,
        out_shape=[
            jax.ShapeDtypeStruct((N, D), jnp.float32),
            jax.ShapeDtypeStruct((N, D), jnp.float32),
        ],
    )(x, agg0, dinv_col, W_self0, b0r, W_neigh1)

    agg1 = _sc_agg(t1, srcp, dstp, cntp)

    out = pl.pallas_call(
        _k5_body,
        out_shape=jax.ShapeDtypeStruct((N, NCLS), jnp.float32),
    )(h1, agg1, dinv_col, W_self1, b1r, fc_W, fcbr)

    return out
